# baseline (speedup 1.0000x reference)
import functools

import jax
import jax.numpy as jnp
from jax.experimental import pallas as pl
from jax.experimental.pallas import tpu as pltpu


def _enc_kernel(terms_sm, cnts_ref, tab_hbm, out_ref, rows, sem, *, BH, L, D):
    j = pl.program_id(0)
    M = BH * L

    for t in range(M):
        idx = terms_sm[j * BH + t // L, t % L]
        pltpu.make_async_copy(
            tab_hbm.at[pl.ds(idx * 128, 128)],
            rows.at[t],
            sem,
        ).start()

    logc = jnp.log(cnts_ref[0].reshape(BH, L, 1))

    for t in range(M):
        pltpu.make_async_copy(
            tab_hbm.at[pl.ds(0, 128)], rows.at[0], sem,
        ).wait()

    G = rows[...].reshape(BH, L, 128)
    w_tok = G[:, :, D:D + 1]
    logits = w_tok + logc
    m = jnp.max(logits, axis=1, keepdims=True)
    e = jnp.exp(logits - m)
    s = jnp.sum(e, axis=1, keepdims=True)
    p = e / s

    out_ref[0] = jnp.sum(p * G, axis=1)


def kernel(terms, cnts, weights_table, embed_table):
    B, L = terms.shape
    V, D = embed_table.shape
    BH = B // 2

    lane = jax.lax.broadcasted_iota(jnp.int32, (V, 128), 1)
    fused = jnp.where(
        lane == D,
        weights_table.astype(jnp.float32),
        jnp.pad(embed_table.astype(jnp.float32), ((0, 0), (0, 128 - D))),
    ).reshape(-1)

    c3 = cnts.astype(jnp.float32).reshape(2, BH * L, 1)

    kernel_fn = functools.partial(_enc_kernel, BH=BH, L=L, D=D)

    out = pl.pallas_call(
        kernel_fn,
        out_shape=jax.ShapeDtypeStruct((2, BH, 128), jnp.float32),
        grid_spec=pltpu.PrefetchScalarGridSpec(
            num_scalar_prefetch=1,
            grid=(2,),
            in_specs=[
                pl.BlockSpec((1, BH * L, 1), lambda j, t: (j, 0, 0)),
                pl.BlockSpec(memory_space=pl.ANY),
            ],
            out_specs=pl.BlockSpec((1, BH, 128), lambda j, t: (j, 0, 0)),
            scratch_shapes=[
                pltpu.VMEM((BH * L, 128), jnp.float32),
                pltpu.SemaphoreType.DMA,
            ],
        ),
        compiler_params=pltpu.CompilerParams(
            dimension_semantics=("parallel",),
            vmem_limit_bytes=32 * 1024 * 1024,
        ),
    )(terms.astype(jnp.int32), c3, fused)

    return out.reshape(B, 128)[:, :D]


# speedup vs baseline: 1.0390x; 1.0390x over previous
import functools

import jax
import jax.numpy as jnp
from jax.experimental import pallas as pl
from jax.experimental.pallas import tpu as pltpu


def _enc_kernel(terms_sm, tvec_ref, cnts_ref, etab_ref, acc_ref, st_ref,
                rows, *, B, L, VH, D):
    j = pl.program_id(0)
    vbase = j * VH

    for t in range(B * L):
        idx = terms_sm[t // L, t % L]
        il = jnp.clip(idx - vbase, 0, VH - 1)
        rows[pl.ds(t, 1), :] = etab_ref[pl.ds(il, 1), :]

    G = rows[...].reshape(B, L, 128)
    tvec = tvec_ref[...]
    keep = (tvec >= vbase) & (tvec < vbase + VH)

    w_tok = G[:, :, D:D + 1]
    logits = jnp.where(keep, w_tok + jnp.log(cnts_ref[...]), -1e30)
    m = jnp.max(logits, axis=1, keepdims=True)
    e = jnp.exp(logits - m)
    s = jnp.sum(e, axis=1, keepdims=True)

    acc_ref[0] = jnp.sum(e * G, axis=1)
    st_ref[0, :, 0:1] = s[:, 0, :]
    st_ref[0, :, 1:2] = m[:, 0, :]


def kernel(terms, cnts, weights_table, embed_table):
    B, L = terms.shape
    V, D = embed_table.shape
    VH = V // 2

    lane = jax.lax.broadcasted_iota(jnp.int32, (V, 128), 1)
    fused = jnp.where(
        lane == D,
        weights_table.astype(jnp.float32),
        jnp.pad(embed_table.astype(jnp.float32), ((0, 0), (0, 128 - D))),
    )

    t3 = terms.astype(jnp.int32).reshape(B, L, 1)
    c3 = cnts.astype(jnp.float32).reshape(B, L, 1)

    kernel_fn = functools.partial(_enc_kernel, B=B, L=L, VH=VH, D=D)

    acc, st = pl.pallas_call(
        kernel_fn,
        out_shape=[
            jax.ShapeDtypeStruct((2, B, 128), jnp.float32),
            jax.ShapeDtypeStruct((2, B, 128), jnp.float32),
        ],
        grid_spec=pltpu.PrefetchScalarGridSpec(
            num_scalar_prefetch=1,
            grid=(2,),
            in_specs=[
                pl.BlockSpec((B, L, 1), lambda j, t: (0, 0, 0)),
                pl.BlockSpec((B, L, 1), lambda j, t: (0, 0, 0)),
                pl.BlockSpec((VH, 128), lambda j, t: (j, 0)),
            ],
            out_specs=[
                pl.BlockSpec((1, B, 128), lambda j, t: (j, 0, 0)),
                pl.BlockSpec((1, B, 128), lambda j, t: (j, 0, 0)),
            ],
            scratch_shapes=[
                pltpu.VMEM((B * L, 128), jnp.float32),
            ],
        ),
        compiler_params=pltpu.CompilerParams(
            dimension_semantics=("parallel",),
            vmem_limit_bytes=32 * 1024 * 1024,
        ),
    )(terms.astype(jnp.int32), t3, c3, fused)

    s0, m0 = st[0, :, 0:1], st[0, :, 1:2]
    s1, m1 = st[1, :, 0:1], st[1, :, 1:2]
    mx = jnp.maximum(m0, m1)
    a0 = jnp.exp(m0 - mx)
    a1 = jnp.exp(m1 - mx)
    den = s0 * a0 + s1 * a1
    num = acc[0] * a0 + acc[1] * a1
    return (num / den)[:, :D]


# speedup vs baseline: 1.5297x; 1.4723x over previous
import functools

import jax
import jax.numpy as jnp
from jax.experimental import pallas as pl
from jax.experimental.pallas import tpu as pltpu


def _enc_kernel(terms_sm, tvec_ref, cnts_ref, wtab_ref, etab_hbm, out_ref,
                erows, wrows, sem, *, BH, L, D):
    j = pl.program_id(0)
    M = BH * L
    base = j * M

    for t in range(M):
        idx = terms_sm[base + t]
        pltpu.make_async_copy(
            etab_hbm.at[pl.ds(idx, 1), :],
            erows.at[pl.ds(t, 1), :],
            sem,
        ).start()

    for t in range(M):
        idx = terms_sm[base + t]
        wrows[pl.ds(t, 1), :] = wtab_ref[pl.ds(idx // 128, 1), :]

    W = wrows[...].reshape(BH, L, 128)
    tvec = tvec_ref[0].reshape(BH, L, 1)

    lane = jax.lax.broadcasted_iota(jnp.int32, (BH, L, 128), 2)
    w_tok = jnp.sum(jnp.where(lane == tvec % 128, W, 0.0),
                    axis=2, keepdims=True)

    logits = w_tok + jnp.log(cnts_ref[0].reshape(BH, L, 1))
    m = jnp.max(logits, axis=1, keepdims=True)
    e = jnp.exp(logits - m)
    s = jnp.sum(e, axis=1, keepdims=True)
    p = e / s

    pltpu.make_async_copy(
        etab_hbm.at[pl.ds(0, M), :], erows.at[pl.ds(0, M), :], sem,
    ).wait()

    G = erows[...].reshape(BH, L, D)
    out_ref[0] = jnp.sum(p * G, axis=1)


def kernel(terms, cnts, weights_table, embed_table):
    B, L = terms.shape
    V, D = embed_table.shape
    BH = B // 2
    NW = V // 128

    wtab = weights_table.astype(jnp.float32).reshape(NW, 128)
    tflat = terms.astype(jnp.int32).reshape(-1)
    t3 = terms.astype(jnp.int32).reshape(2, BH * L, 1)
    c3 = cnts.astype(jnp.float32).reshape(2, BH * L, 1)

    kernel_fn = functools.partial(_enc_kernel, BH=BH, L=L, D=D)

    out = pl.pallas_call(
        kernel_fn,
        out_shape=jax.ShapeDtypeStruct((2, BH, D), jnp.float32),
        grid_spec=pltpu.PrefetchScalarGridSpec(
            num_scalar_prefetch=1,
            grid=(2,),
            in_specs=[
                pl.BlockSpec((1, BH * L, 1), lambda j, t: (j, 0, 0)),
                pl.BlockSpec((1, BH * L, 1), lambda j, t: (j, 0, 0)),
                pl.BlockSpec((NW, 128), lambda j, t: (0, 0)),
                pl.BlockSpec(memory_space=pl.ANY),
            ],
            out_specs=pl.BlockSpec((1, BH, D), lambda j, t: (j, 0, 0)),
            scratch_shapes=[
                pltpu.VMEM((BH * L, D), jnp.float32),
                pltpu.VMEM((BH * L, 128), jnp.float32),
                pltpu.SemaphoreType.DMA,
            ],
        ),
        compiler_params=pltpu.CompilerParams(
            dimension_semantics=("parallel",),
            vmem_limit_bytes=32 * 1024 * 1024,
        ),
    )(tflat, t3, c3, wtab, embed_table.astype(jnp.float32))

    return out.reshape(B, D)
